# Initial kernel scaffold; baseline (speedup 1.0000x reference)
import jax
import jax.numpy as jnp
from jax.experimental import pallas as pl


def kernel(src, tgt, src_embedding, tgt_embedding):
    raise NotImplementedError("write your pallas kernel here")


# speedup vs baseline: 1.3179x; 1.3179x over previous
import functools

import jax
import jax.numpy as jnp
from jax import lax
from jax.experimental import pallas as pl
from jax.experimental.pallas import tpu as pltpu
from jax.experimental.pallas import tpu_sc as plsc

B = 16
C = 256
N = 4096
K = 512
L = 16
RADIX = 32
DIGIT_BITS = 5
NUM_PASSES = 7
SCAN_BASE = 1
CSLAB = 8


def _norm_body(src_ref, tgt_ref, ns_ref, nt_ref):
    x = src_ref[0]
    nx = jnp.sqrt(jnp.sum(x * x, axis=0))
    ns_ref[0, 0, :] = jnp.bitwise_not(lax.bitcast_convert_type(nx, jnp.int32))
    y = tgt_ref[0]
    ny = jnp.sqrt(jnp.sum(y * y, axis=0))
    nt_ref[0, 0, :] = jnp.bitwise_not(lax.bitcast_convert_type(ny, jnp.int32))


_norms_call = pl.pallas_call(
    _norm_body,
    grid=(B,),
    in_specs=[
        pl.BlockSpec((1, C, N), lambda b: (b, 0, 0)),
        pl.BlockSpec((1, C, N), lambda b: (b, 0, 0)),
    ],
    out_specs=[
        pl.BlockSpec((1, 1, N), lambda b: (b, 0, 0)),
        pl.BlockSpec((1, 1, N), lambda b: (b, 0, 0)),
    ],
    out_shape=[
        jax.ShapeDtypeStruct((B, 1, N), jnp.int32),
        jax.ShapeDtypeStruct((B, 1, N), jnp.int32),
    ],
)


def _digit(k, shift):
    if shift:
        k = lax.shift_right_logical(k, jnp.full((L,), shift, jnp.int32))
    return jnp.bitwise_and(k, RADIX - 1)


_SC_MESH = plsc.VectorSubcoreMesh(core_axis_name="c", subcore_axis_name="s")
_SC_PARAMS = pltpu.CompilerParams(needs_layout_passes=False)


@functools.partial(
    pl.kernel,
    out_type=[
        jax.ShapeDtypeStruct((2, B, K), jnp.int32),
        jax.ShapeDtypeStruct((2, B, 3, K), jnp.float32),
    ],
    mesh=_SC_MESH,
    compiler_params=_SC_PARAMS,
    scratch_types=[
        pltpu.VMEM((N,), jnp.int32),
        pltpu.VMEM((N,), jnp.int32),
        pltpu.VMEM((N,), jnp.int32),
        pltpu.VMEM((N,), jnp.int32),
        pltpu.VMEM((RADIX,), jnp.int32),
        pltpu.VMEM((3, N), jnp.float32),
        pltpu.VMEM((3, K), jnp.float32),
    ],
)
def _sc_sort(keys_all, kp_all, idx_out, kp_out,
             keys0, keys1, vals0, vals1, hist, kp_stage, kp_buf):
    c = lax.axis_index("c")
    s = lax.axis_index("s")
    lanes = lax.iota(jnp.int32, L)
    nvec = N // L

    pltpu.sync_copy(keys_all.at[c, s], keys0)

    def zero_hist():
        z = jnp.zeros((L,), jnp.int32)
        hist[pl.ds(0, L)] = z
        hist[pl.ds(L, L)] = z

    def spread_offsets():
        h0 = hist[pl.ds(0, L)]
        h1 = hist[pl.ds(L, L)]
        c0 = plsc.cumsum(h0)
        c1 = plsc.cumsum(h1)
        t0 = jnp.sum(h0)
        hist[pl.ds(0, L)] = c0 - h0
        hist[pl.ds(L, L)] = c1 - h1 + t0

    def hist_add(d, cnt, last):
        plsc.addupdate_scatter(hist, [d], cnt + (1 - SCAN_BASE), mask=last)

    zero_hist()

    def p0_count(i, carry):
        d = _digit(keys0[pl.ds(i * L, L)], 0)
        cnt, last = plsc.scan_count(d)
        hist_add(d, cnt, last)
        return carry

    lax.fori_loop(0, nvec, p0_count, 0)
    spread_offsets()

    def p0_perm(i, carry):
        key = keys0[pl.ds(i * L, L)]
        val = lanes + i * L
        d = _digit(key, 0)
        cnt, last = plsc.scan_count(d)
        base = plsc.load_gather(hist, [d])
        pos = base + cnt - SCAN_BASE
        plsc.store_scatter(keys1, [pos], key)
        plsc.store_scatter(vals1, [pos], val)
        hist_add(d, cnt, last)
        return carry

    lax.fori_loop(0, nvec, p0_perm, 0)

    for p in range(1, NUM_PASSES):
        shift = p * DIGIT_BITS
        kin, vin, kout, vout = (
            (keys1, vals1, keys0, vals0) if p % 2 else (keys0, vals0, keys1, vals1)
        )
        zero_hist()

        def p_count(i, carry, kin=kin, shift=shift):
            d = _digit(kin[pl.ds(i * L, L)], shift)
            cnt, last = plsc.scan_count(d)
            hist_add(d, cnt, last)
            return carry

        lax.fori_loop(0, nvec, p_count, 0)
        spread_offsets()

        def p_perm(i, carry, kin=kin, vin=vin, kout=kout, vout=vout, shift=shift):
            key = kin[pl.ds(i * L, L)]
            val = vin[pl.ds(i * L, L)]
            d = _digit(key, shift)
            cnt, last = plsc.scan_count(d)
            base = plsc.load_gather(hist, [d])
            pos = base + cnt - SCAN_BASE
            plsc.store_scatter(kout, [pos], key)
            plsc.store_scatter(vout, [pos], val)
            hist_add(d, cnt, last)
            return carry

        lax.fori_loop(0, nvec, p_perm, 0)

    sorted_vals = vals0 if NUM_PASSES % 2 else vals1

    pltpu.sync_copy(kp_all.at[c, s], kp_stage)
    for v in range(K // L):
        iv = sorted_vals[pl.ds(v * L, L)]
        for ch in range(3):
            g = plsc.load_gather(kp_stage, [jnp.full((L,), ch, jnp.int32), iv])
            kp_buf[ch, pl.ds(v * L, L)] = g
    pltpu.sync_copy(kp_buf, kp_out.at[c, s])
    pltpu.sync_copy(sorted_vals.at[pl.ds(0, K)], idx_out.at[c, s])


@functools.partial(
    pl.kernel,
    out_type=[
        jax.ShapeDtypeStruct((B, C, K), jnp.float32),
        jax.ShapeDtypeStruct((B, C, K), jnp.float32),
    ],
    mesh=_SC_MESH,
    compiler_params=_SC_PARAMS,
    scratch_types=[
        pltpu.VMEM((CSLAB, N), jnp.float32),
        pltpu.VMEM((CSLAB, K), jnp.float32),
        pltpu.VMEM((K,), jnp.int32),
    ],
)
def _sc_gather(idx_all, semb, temb, semb_out, temb_out,
               emb_stage, emb_buf, idx_v):
    c = lax.axis_index("c")
    s = lax.axis_index("s")
    wid = c * 16 + s
    ch0 = wid * CSLAB

    def do_side(emb, emb_out, side):
        def row_body(b, carry):
            pltpu.sync_copy(idx_all.at[side, b], idx_v)
            pltpu.sync_copy(emb.at[b, pl.ds(ch0, CSLAB), :], emb_stage)
            for ch in range(CSLAB):
                chv = jnp.full((L,), ch, jnp.int32)
                for v in range(K // L):
                    iv = idx_v[pl.ds(v * L, L)]
                    emb_buf[ch, pl.ds(v * L, L)] = plsc.load_gather(
                        emb_stage, [chv, iv])
            pltpu.sync_copy(emb_buf, emb_out.at[b, pl.ds(ch0, CSLAB), :])
            return carry

        lax.fori_loop(0, B, row_body, 0)

    do_side(semb, semb_out, 0)
    do_side(temb, temb_out, 1)


def kernel(src, tgt, src_embedding, tgt_embedding):
    ns, nt = _norms_call(src_embedding, tgt_embedding)
    keys_all = jnp.stack([ns.reshape(B, N), nt.reshape(B, N)])
    kp_all = jnp.stack([src, tgt])
    idx_all, kp_out = _sc_sort(keys_all, kp_all)
    semb_out, temb_out = _sc_gather(idx_all, src_embedding, tgt_embedding)
    return kp_out[0], kp_out[1], semb_out, temb_out
